# baseline (speedup 1.0000x reference)
import jax
import jax.numpy as jnp
from jax.experimental import pallas as pl

ALPHA = 8


def _pack_body(x_ref, slow_ref, fast_ref):
    g = pl.program_id(1)
    fast_ref[...] = x_ref[...]
    slow_ref[0, 0] = x_ref[0, g]


def kernel(x):
    C, T, H, W = x.shape
    G = T // ALPHA
    SB, LN = (H * W) // 128, 128
    xf = x.reshape(C, T, SB, LN)
    slow, fast = pl.pallas_call(
        _pack_body,
        grid=(C, G),
        in_specs=[
            pl.BlockSpec((1, ALPHA, SB, LN), lambda c, g: (c, g, 0, 0)),
        ],
        out_specs=[
            pl.BlockSpec((1, 1, SB, LN), lambda c, g: (c, g, 0, 0)),
            pl.BlockSpec((1, ALPHA, SB, LN), lambda c, g: (c, g, 0, 0)),
        ],
        out_shape=[
            jax.ShapeDtypeStruct((C, G, SB, LN), x.dtype),
            jax.ShapeDtypeStruct((C, T, SB, LN), x.dtype),
        ],
    )(xf)
    return (slow.reshape(C, G, H, W), fast.reshape(C, T, H, W))


# speedup vs baseline: 4.3726x; 4.3726x over previous
import jax
import jax.numpy as jnp
from jax.experimental import pallas as pl

ALPHA = 8
FRAMES = 16


def _pack_body(x_ref, slow_ref, fast_ref):
    h = pl.program_id(1)
    fast_ref[...] = x_ref[...]
    slow_ref[0, 0] = x_ref[0, 2 * h]
    slow_ref[0, 1] = x_ref[0, 2 * h + 9]


def kernel(x):
    C, T, H, W = x.shape
    G = T // ALPHA
    NG = T // FRAMES
    slow, fast = pl.pallas_call(
        _pack_body,
        grid=(C, NG),
        in_specs=[
            pl.BlockSpec((1, FRAMES, H, W), lambda c, h: (c, h, 0, 0)),
        ],
        out_specs=[
            pl.BlockSpec((1, 2, H, W), lambda c, h: (c, h, 0, 0)),
            pl.BlockSpec((1, FRAMES, H, W), lambda c, h: (c, h, 0, 0)),
        ],
        out_shape=[
            jax.ShapeDtypeStruct((C, G, H, W), x.dtype),
            jax.ShapeDtypeStruct((C, T, H, W), x.dtype),
        ],
    )(x)
    return (slow, fast)


# speedup vs baseline: 4.6305x; 1.0590x over previous
import jax
import jax.numpy as jnp
from jax.experimental import pallas as pl

ALPHA = 8
FRAMES = 32


def _pack_body(x_ref, slow_ref, fast_ref):
    h = pl.program_id(1)
    fast_ref[...] = x_ref[...]
    for j in range(4):
        slow_ref[0, j] = x_ref[0, 4 * h + 9 * j]


def kernel(x):
    C, T, H, W = x.shape
    G = T // ALPHA
    NG = T // FRAMES
    slow, fast = pl.pallas_call(
        _pack_body,
        grid=(C, NG),
        in_specs=[
            pl.BlockSpec((1, FRAMES, H, W), lambda c, h: (c, h, 0, 0)),
        ],
        out_specs=[
            pl.BlockSpec((1, 4, H, W), lambda c, h: (c, h, 0, 0)),
            pl.BlockSpec((1, FRAMES, H, W), lambda c, h: (c, h, 0, 0)),
        ],
        out_shape=[
            jax.ShapeDtypeStruct((C, G, H, W), x.dtype),
            jax.ShapeDtypeStruct((C, T, H, W), x.dtype),
        ],
    )(x)
    return (slow, fast)
